# Initial kernel scaffold; baseline (speedup 1.0000x reference)
import jax
import jax.numpy as jnp
from jax.experimental import pallas as pl


def kernel(x):
    raise NotImplementedError("write your pallas kernel here")


# speedup vs baseline: 53.8324x; 53.8324x over previous
import functools

import jax
import jax.numpy as jnp
from jax import lax
from jax.experimental import pallas as pl
from jax.experimental.pallas import tpu as pltpu
from jax.experimental.pallas import tpu_sc as plsc

B = 8
C_, H_, W_ = 96, 224, 224
ROW = C_ * H_ * W_
K = int(0.1 * ROW)
NC, NS, L = 2, 16, 16
NW = NC * NS
W_ELEMS = (B * ROW) // NW
CHUNK = 8192
NBUF = 3
N_CHUNKS = W_ELEMS // CHUNK
assert W_ELEMS % CHUNK == 0 and N_CHUNKS % NBUF == 0


def _make_hist_pass(stage):
    nbins = 4096 if stage < 3 else 256
    hist_words = nbins * L
    has_pref = stage > 1

    scratch = [pltpu.VMEM((CHUNK,), jnp.int32) for _ in range(NBUF)]
    scratch += [
        pltpu.VMEM((hist_words,), jnp.int32),
        pltpu.VMEM((nbins,), jnp.int32),
    ]
    if has_pref:
        scratch.append(pltpu.VMEM((128,), jnp.int32))
    scratch += [pltpu.SemaphoreType.DMA for _ in range(NBUF)]

    mesh = plsc.VectorSubcoreMesh(
        core_axis_name="c", subcore_axis_name="s",
        num_cores=NC, num_subcores=NS)

    def body(*refs):
        if has_pref:
            (x_hbm, pref_hbm, out_hbm, b0, b1, b2, histv, histr, prefbuf,
             s0, s1, s2) = refs
        else:
            (x_hbm, out_hbm, b0, b1, b2, histv, histr, s0, s1, s2) = refs
        bufs, sems = (b0, b1, b2), (s0, s1, s2)

        cid = lax.axis_index("c")
        sid = lax.axis_index("s")
        wid = sid * NC + cid
        base = wid * W_ELEMS
        lane = lax.iota(jnp.int32, L)
        ones = jnp.ones((L,), jnp.int32)
        zeros16 = jnp.zeros((L,), jnp.int32)

        @plsc.parallel_loop(0, hist_words, step=L, unroll=8)
        def _(i):
            histv[pl.ds(i, L)] = zeros16

        if has_pref:
            pltpu.sync_copy(pref_hbm.at[wid], prefbuf)
            pv = prefbuf[pl.ds(0, L)]

        for b in range(NBUF):
            pltpu.make_async_copy(
                x_hbm.at[pl.ds(base + b * CHUNK, CHUNK)], bufs[b], sems[b]
            ).start()

        def process(buf):
            @plsc.parallel_loop(0, CHUNK, step=L, unroll=8)
            def _(e):
                iv = buf[pl.ds(e, L)]
                if stage == 1:
                    idx = ((iv >> 16) & 0xFFF0) + lane
                    m = None
                elif stage == 2:
                    m = (iv >> 20) == pv
                    idx = ((iv >> 4) & 0xFFF0) + lane
                else:
                    m = (iv >> 8) == pv
                    idx = ((iv << 4) & 0xFF0) + lane
                plsc.addupdate_scatter(histv, [idx], ones, mask=m)

        def outer(i, carry):
            for b in range(NBUF):
                c = i * NBUF + b
                off = base + c * CHUNK
                pltpu.make_async_copy(
                    x_hbm.at[pl.ds(off, CHUNK)], bufs[b], sems[b]).wait()
                process(bufs[b])
                nxt = c + NBUF

                @pl.when(nxt < N_CHUNKS)
                def _():
                    pltpu.make_async_copy(
                        x_hbm.at[pl.ds(base + nxt * CHUNK, CHUNK)],
                        bufs[b], sems[b]).start()
            return carry
        lax.fori_loop(0, N_CHUNKS // NBUF, outer, 0)

        lane16 = lane * L

        @plsc.parallel_loop(0, nbins, step=L, unroll=2)
        def _(g):
            basei = g * L
            acc = zeros16
            for l in range(L):
                acc = acc + plsc.load_gather(histv, [basei + lane16 + l])
            histr[pl.ds(g, L)] = acc

        pltpu.sync_copy(histr, out_hbm.at[wid])

    return functools.partial(
        pl.kernel, body,
        out_type=jax.ShapeDtypeStruct((NW, nbins), jnp.int32),
        mesh=mesh, scratch_types=scratch,
        compiler_params=pltpu.CompilerParams(
            needs_layout_passes=False, use_tc_tiling_on_sc=False),
        name=f"kwta_hist{stage}")()


def _tri(n, lower):
    i = lax.broadcasted_iota(jnp.int32, (n, n), 0)
    j = lax.broadcasted_iota(jnp.int32, (n, n), 1)
    return ((i < j) if lower else (i > j)).astype(jnp.float32)


def _matmul(a, b):
    return lax.dot_general(a, b, (((1,), (0,)), ((), ())),
                           precision=lax.Precision.HIGHEST,
                           preferred_element_type=jnp.float32)


def _select_body(stage, *refs):
    nbins = 4096 if stage < 3 else 256
    nch = nbins // 128
    if stage == 1:
        h_ref, pref_out, carry_out = refs
        carry_ref = None
    elif stage == 2:
        h_ref, carry_ref, pref_out, carry_out = refs
    else:
        h_ref, carry_ref, t_out = refs

    h = h_ref[...].astype(jnp.float32)
    hr = jnp.sum(h.reshape(B, NW // B, nbins), axis=1)
    h3 = hr.reshape(B, nch, 128)
    cs = jnp.sum(h3, axis=2)

    wl_up = _matmul(h3.reshape(B * nch, 128), _tri(128, False)
                    ).reshape(B, nch, 128)
    wl_dn = _matmul(h3.reshape(B * nch, 128), _tri(128, True)
                    ).reshape(B, nch, 128)

    if stage == 1:
        ic = lax.broadcasted_iota(jnp.int32, (nch, nch), 0)
        jc = lax.broadcasted_iota(jnp.int32, (nch, nch), 1)
        posrow, poscol = ic < nch // 2, jc < nch // 2
        mc = ((poscol & posrow & (ic > jc))
              | ((~poscol) & (posrow | ((~posrow) & (ic < jc))))
              ).astype(jnp.float32)
        gc = _matmul(cs, mc)
        chunkpos = (lax.broadcasted_iota(jnp.int32, (1, nch, 1), 1)
                    < nch // 2)
        g = gc[:, :, None] + jnp.where(chunkpos, wl_up, wl_dn)
        kv = jnp.full((B, 1, 1), float(K), jnp.float32)
    else:
        gc_up = _matmul(cs, _tri(nch, False))
        gc_dn = _matmul(cs, _tri(nch, True))
        prev = carry_ref[...][:, 0:1]
        negrow = (prev >= (2048 if stage == 2 else 0x800000))[:, :, None]
        g = jnp.where(negrow, gc_dn[:, :, None] + wl_dn,
                      gc_up[:, :, None] + wl_up)
        kv = carry_ref[...][:, 1:2].astype(jnp.float32)[:, :, None]

    cond = (g < kv) & (g + h3 >= kv)
    bidx = (lax.broadcasted_iota(jnp.int32, (B, nch, 128), 1) * 128
            + lax.broadcasted_iota(jnp.int32, (B, nch, 128), 2)
            ).astype(jnp.float32)
    bsel = jnp.sum(jnp.sum(jnp.where(cond, bidx, 0.0), axis=2),
                   axis=1, keepdims=True)
    kres = jnp.sum(jnp.sum(jnp.where(cond, kv - g, 0.0), axis=2),
                   axis=1, keepdims=True)
    bi = bsel.astype(jnp.int32)
    ki = kres.astype(jnp.int32)

    if stage == 3:
        ut = carry_ref[...][:, 0:1] * 256 + bi
        t_out[...] = jnp.broadcast_to(
            lax.bitcast_convert_type(ut, jnp.float32), (B, 128))
        return
    if stage == 1:
        newraw = bi
        sext = jnp.where(bi >= 2048, bi - 4096, bi)
    else:
        newraw = carry_ref[...][:, 0:1] * 4096 + bi
        sext = jnp.where(newraw >= 0x800000, newraw - 0x1000000, newraw)

    pref_out[...] = jnp.broadcast_to(
        sext[:, None, :], (B, NW // B, 128)).reshape(NW, 128)
    ci = lax.broadcasted_iota(jnp.int32, (B, 128), 1)
    carry_out[...] = jnp.where(
        ci == 0, jnp.broadcast_to(newraw, (B, 128)),
        jnp.where(ci == 1, jnp.broadcast_to(ki, (B, 128)), 0))


def _select(stage, *args):
    if stage < 3:
        out_shape = (jax.ShapeDtypeStruct((NW, 128), jnp.int32),
                     jax.ShapeDtypeStruct((B, 128), jnp.int32))
    else:
        out_shape = jax.ShapeDtypeStruct((B, 128), jnp.float32)
    return pl.pallas_call(
        functools.partial(_select_body, stage),
        out_shape=out_shape,
        name=f"kwta_select{stage}")(*args)


_MROWS, _MCOLS = 147, 32768
_MBLK = 4096
assert _MROWS * _MCOLS == ROW and _MCOLS % _MBLK == 0


def _mask_body(t_ref, x_ref, o_ref):
    tv = t_ref[pl.program_id(0), 0]
    xb = x_ref[...]
    o_ref[...] = jnp.where(xb >= tv, xb, 0.0)


def _mask(x3d, t):
    return pl.pallas_call(
        _mask_body,
        grid=(B, _MCOLS // _MBLK),
        in_specs=[
            pl.BlockSpec((B, 128), lambda r, c: (0, 0)),
            pl.BlockSpec((1, _MROWS, _MBLK), lambda r, c: (r, 0, c)),
        ],
        out_specs=pl.BlockSpec((1, _MROWS, _MBLK), lambda r, c: (r, 0, c)),
        out_shape=jax.ShapeDtypeStruct((B, _MROWS, _MCOLS), jnp.float32),
        name="kwta_mask")(t, x3d)


def kernel(x):
    xf = lax.bitcast_convert_type(x, jnp.int32).reshape(-1)
    h1 = _make_hist_pass(1)(xf)
    p1, c1 = _select(1, h1)
    h2 = _make_hist_pass(2)(xf, p1)
    p2, c2 = _select(2, h2, c1)
    h3 = _make_hist_pass(3)(xf, p2)
    t = _select(3, h3, c2)
    out = _mask(x.reshape(B, _MROWS, _MCOLS), t)
    return out.reshape(x.shape)


# speedup vs baseline: 102.1985x; 1.8985x over previous
import functools

import jax
import jax.numpy as jnp
from jax import lax
from jax.experimental import pallas as pl
from jax.experimental.pallas import tpu as pltpu
from jax.experimental.pallas import tpu_sc as plsc

B = 8
C_, H_, W_ = 96, 224, 224
ROW = C_ * H_ * W_
K = int(0.1 * ROW)
NC, NS, L = 2, 16, 16
NW = NC * NS
W_ELEMS = (B * ROW) // NW
CHUNK = 8192
NBUF = 3
N_CHUNKS = W_ELEMS // CHUNK
assert W_ELEMS % CHUNK == 0 and N_CHUNKS % NBUF == 0


def _make_hist_pass(stage):
    nbins = 4096 if stage < 3 else 256
    hist_words = nbins * L
    has_pref = stage > 1

    scratch = [pltpu.VMEM((CHUNK,), jnp.float32) for _ in range(NBUF)]
    scratch += [
        pltpu.VMEM((hist_words,), jnp.int32),
        pltpu.VMEM((nbins,), jnp.int32),
    ]
    if has_pref:
        scratch.append(pltpu.VMEM((128,), jnp.int32))
    scratch += [pltpu.SemaphoreType.DMA for _ in range(NBUF)]

    mesh = plsc.VectorSubcoreMesh(
        core_axis_name="c", subcore_axis_name="s",
        num_cores=NC, num_subcores=NS)

    def body(*refs):
        if has_pref:
            (x_hbm, pref_hbm, out_hbm, b0, b1, b2, histv, histr, prefbuf,
             s0, s1, s2) = refs
        else:
            (x_hbm, out_hbm, b0, b1, b2, histv, histr, s0, s1, s2) = refs
        bufs, sems = (b0, b1, b2), (s0, s1, s2)

        cid = lax.axis_index("c")
        sid = lax.axis_index("s")
        wid = sid * NC + cid
        base = wid * W_ELEMS
        lane = lax.iota(jnp.int32, L)
        ones = jnp.ones((L,), jnp.int32)
        zeros16 = jnp.zeros((L,), jnp.int32)

        @plsc.parallel_loop(0, hist_words, step=L, unroll=8)
        def _(i):
            histv[pl.ds(i, L)] = zeros16

        if has_pref:
            pltpu.sync_copy(pref_hbm.at[wid], prefbuf)
            pv = prefbuf[pl.ds(0, L)]

        for b in range(NBUF):
            pltpu.make_async_copy(
                x_hbm.at[pl.ds(base + b * CHUNK, CHUNK)], bufs[b], sems[b]
            ).start()

        def process(buf):
            @plsc.parallel_loop(0, CHUNK, step=L, unroll=8)
            def _(e):
                iv = plsc.bitcast(buf[pl.ds(e, L)], jnp.int32)
                if stage == 1:
                    idx = ((iv >> 16) & 0xFFF0) + lane
                    m = None
                elif stage == 2:
                    m = (iv >> 20) == pv
                    idx = ((iv >> 4) & 0xFFF0) + lane
                else:
                    m = (iv >> 8) == pv
                    idx = ((iv << 4) & 0xFF0) + lane
                plsc.addupdate_scatter(histv, [idx], ones, mask=m)

        def outer(i, carry):
            for b in range(NBUF):
                c = i * NBUF + b
                off = base + c * CHUNK
                pltpu.make_async_copy(
                    x_hbm.at[pl.ds(off, CHUNK)], bufs[b], sems[b]).wait()
                process(bufs[b])
                nxt = c + NBUF

                @pl.when(nxt < N_CHUNKS)
                def _():
                    pltpu.make_async_copy(
                        x_hbm.at[pl.ds(base + nxt * CHUNK, CHUNK)],
                        bufs[b], sems[b]).start()
            return carry
        lax.fori_loop(0, N_CHUNKS // NBUF, outer, 0)

        lane16 = lane * L

        @plsc.parallel_loop(0, nbins, step=L, unroll=2)
        def _(g):
            basei = g * L
            acc = zeros16
            for l in range(L):
                acc = acc + plsc.load_gather(histv, [basei + lane16 + l])
            histr[pl.ds(g, L)] = acc

        pltpu.sync_copy(histr, out_hbm.at[wid])

    return functools.partial(
        pl.kernel, body,
        out_type=jax.ShapeDtypeStruct((NW, nbins), jnp.int32),
        mesh=mesh, scratch_types=scratch,
        compiler_params=pltpu.CompilerParams(
            needs_layout_passes=False, use_tc_tiling_on_sc=False),
        name=f"kwta_hist{stage}")()


def _tri(n, lower):
    i = lax.broadcasted_iota(jnp.int32, (n, n), 0)
    j = lax.broadcasted_iota(jnp.int32, (n, n), 1)
    return ((i < j) if lower else (i > j)).astype(jnp.float32)


def _matmul(a, b):
    return lax.dot_general(a, b, (((1,), (0,)), ((), ())),
                           precision=lax.Precision.HIGHEST,
                           preferred_element_type=jnp.float32)


def _select_body(stage, *refs):
    nbins = 4096 if stage < 3 else 256
    nch = nbins // 128
    if stage == 1:
        h_ref, pref_out, carry_out = refs
        carry_ref = None
    elif stage == 2:
        h_ref, carry_ref, pref_out, carry_out = refs
    else:
        h_ref, carry_ref, t_out = refs

    h = h_ref[...].astype(jnp.float32)
    hr = jnp.sum(h.reshape(B, NW // B, nbins), axis=1)
    h3 = hr.reshape(B, nch, 128)
    cs = jnp.sum(h3, axis=2)

    wl_up = _matmul(h3.reshape(B * nch, 128), _tri(128, False)
                    ).reshape(B, nch, 128)
    wl_dn = _matmul(h3.reshape(B * nch, 128), _tri(128, True)
                    ).reshape(B, nch, 128)

    if stage == 1:
        ic = lax.broadcasted_iota(jnp.int32, (nch, nch), 0)
        jc = lax.broadcasted_iota(jnp.int32, (nch, nch), 1)
        posrow, poscol = ic < nch // 2, jc < nch // 2
        mc = ((poscol & posrow & (ic > jc))
              | ((~poscol) & (posrow | ((~posrow) & (ic < jc))))
              ).astype(jnp.float32)
        gc = _matmul(cs, mc)
        chunkpos = (lax.broadcasted_iota(jnp.int32, (1, nch, 1), 1)
                    < nch // 2)
        g = gc[:, :, None] + jnp.where(chunkpos, wl_up, wl_dn)
        kv = jnp.full((B, 1, 1), float(K), jnp.float32)
    else:
        gc_up = _matmul(cs, _tri(nch, False))
        gc_dn = _matmul(cs, _tri(nch, True))
        prev = carry_ref[...][:, 0:1]
        negrow = (prev >= (2048 if stage == 2 else 0x800000))[:, :, None]
        g = jnp.where(negrow, gc_dn[:, :, None] + wl_dn,
                      gc_up[:, :, None] + wl_up)
        kv = carry_ref[...][:, 1:2].astype(jnp.float32)[:, :, None]

    cond = (g < kv) & (g + h3 >= kv)
    bidx = (lax.broadcasted_iota(jnp.int32, (B, nch, 128), 1) * 128
            + lax.broadcasted_iota(jnp.int32, (B, nch, 128), 2)
            ).astype(jnp.float32)
    bsel = jnp.sum(jnp.sum(jnp.where(cond, bidx, 0.0), axis=2),
                   axis=1, keepdims=True)
    kres = jnp.sum(jnp.sum(jnp.where(cond, kv - g, 0.0), axis=2),
                   axis=1, keepdims=True)
    bi = bsel.astype(jnp.int32)
    ki = kres.astype(jnp.int32)

    if stage == 3:
        ut = carry_ref[...][:, 0:1] * 256 + bi
        t_out[...] = jnp.broadcast_to(
            lax.bitcast_convert_type(ut, jnp.float32), (B, 128))
        return
    if stage == 1:
        newraw = bi
        sext = jnp.where(bi >= 2048, bi - 4096, bi)
    else:
        newraw = carry_ref[...][:, 0:1] * 4096 + bi
        sext = jnp.where(newraw >= 0x800000, newraw - 0x1000000, newraw)

    pref_out[...] = jnp.broadcast_to(
        sext[:, None, :], (B, NW // B, 128)).reshape(NW, 128)
    ci = lax.broadcasted_iota(jnp.int32, (B, 128), 1)
    carry_out[...] = jnp.where(
        ci == 0, jnp.broadcast_to(newraw, (B, 128)),
        jnp.where(ci == 1, jnp.broadcast_to(ki, (B, 128)), 0))


def _select(stage, *args):
    if stage < 3:
        out_shape = (jax.ShapeDtypeStruct((NW, 128), jnp.int32),
                     jax.ShapeDtypeStruct((B, 128), jnp.int32))
    else:
        out_shape = jax.ShapeDtypeStruct((B, 128), jnp.float32)
    return pl.pallas_call(
        functools.partial(_select_body, stage),
        out_shape=out_shape,
        name=f"kwta_select{stage}")(*args)


_MC = 12
assert C_ % _MC == 0


def _mask_body(t_ref, x_ref, o_ref):
    tv = t_ref[pl.program_id(0), 0]
    xb = x_ref[...]
    o_ref[...] = jnp.where(xb >= tv, xb, 0.0)


def _mask(x, t):
    return pl.pallas_call(
        _mask_body,
        grid=(B, C_ // _MC),
        in_specs=[
            pl.BlockSpec((B, 128), lambda r, c: (0, 0)),
            pl.BlockSpec((1, _MC, H_, W_), lambda r, c: (r, c, 0, 0)),
        ],
        out_specs=pl.BlockSpec((1, _MC, H_, W_), lambda r, c: (r, c, 0, 0)),
        out_shape=jax.ShapeDtypeStruct((B, C_, H_, W_), jnp.float32),
        name="kwta_mask")(t, x)


def kernel(x):
    xf = x.reshape(-1)
    h1 = _make_hist_pass(1)(xf)
    p1, c1 = _select(1, h1)
    h2 = _make_hist_pass(2)(xf, p1)
    p2, c2 = _select(2, h2, c1)
    h3 = _make_hist_pass(3)(xf, p2)
    t = _select(3, h3, c2)
    return _mask(x, t)


# speedup vs baseline: 103.1786x; 1.0096x over previous
import functools

import jax
import jax.numpy as jnp
from jax import lax
from jax.experimental import pallas as pl
from jax.experimental.pallas import tpu as pltpu
from jax.experimental.pallas import tpu_sc as plsc

B = 8
C_, H_, W_ = 96, 224, 224
ROW = C_ * H_ * W_
K = int(0.1 * ROW)
NC, NS, L = 2, 16, 16
NW = NC * NS
W_ELEMS = (B * ROW) // NW
CHUNK = 8192
NBUF = 3
N_CHUNKS = W_ELEMS // CHUNK
assert W_ELEMS % CHUNK == 0 and N_CHUNKS % NBUF == 0


def _make_hist_pass(stage):
    nbins = 4096 if stage < 3 else 256
    hist_words = nbins * L
    has_pref = stage > 1

    scratch = [pltpu.VMEM((CHUNK,), jnp.float32) for _ in range(NBUF)]
    scratch += [
        pltpu.VMEM((hist_words,), jnp.int32),
        pltpu.VMEM((nbins,), jnp.int32),
    ]
    if has_pref:
        scratch.append(pltpu.VMEM((128,), jnp.int32))
    scratch += [pltpu.SemaphoreType.DMA for _ in range(NBUF)]

    mesh = plsc.VectorSubcoreMesh(
        core_axis_name="c", subcore_axis_name="s",
        num_cores=NC, num_subcores=NS)

    def body(*refs):
        if has_pref:
            (x_hbm, pref_hbm, out_hbm, b0, b1, b2, histv, histr, prefbuf,
             s0, s1, s2) = refs
        else:
            (x_hbm, out_hbm, b0, b1, b2, histv, histr, s0, s1, s2) = refs
        bufs, sems = (b0, b1, b2), (s0, s1, s2)

        cid = lax.axis_index("c")
        sid = lax.axis_index("s")
        wid = sid * NC + cid
        base = wid * W_ELEMS
        lane = lax.iota(jnp.int32, L)
        ones = jnp.ones((L,), jnp.int32)
        zeros16 = jnp.zeros((L,), jnp.int32)

        @plsc.parallel_loop(0, hist_words, step=L, unroll=8)
        def _(i):
            histv[pl.ds(i, L)] = zeros16

        if has_pref:
            pltpu.sync_copy(pref_hbm.at[wid], prefbuf)
            pv = prefbuf[pl.ds(0, L)]

        for b in range(NBUF):
            pltpu.make_async_copy(
                x_hbm.at[pl.ds(base + b * CHUNK, CHUNK)], bufs[b], sems[b]
            ).start()

        def process(buf):
            @plsc.parallel_loop(0, CHUNK, step=L, unroll=16)
            def _(e):
                iv = plsc.bitcast(buf[pl.ds(e, L)], jnp.int32)
                if stage == 1:
                    idx = ((iv >> 16) & 0xFFF0) + lane
                    m = None
                elif stage == 2:
                    m = (iv >> 20) == pv
                    idx = ((iv >> 4) & 0xFFF0) + lane
                else:
                    m = (iv >> 8) == pv
                    idx = ((iv << 4) & 0xFF0) + lane
                plsc.addupdate_scatter(histv, [idx], ones, mask=m)

        def outer(i, carry):
            for b in range(NBUF):
                c = i * NBUF + b
                off = base + c * CHUNK
                pltpu.make_async_copy(
                    x_hbm.at[pl.ds(off, CHUNK)], bufs[b], sems[b]).wait()
                process(bufs[b])
                nxt = c + NBUF

                @pl.when(nxt < N_CHUNKS)
                def _():
                    pltpu.make_async_copy(
                        x_hbm.at[pl.ds(base + nxt * CHUNK, CHUNK)],
                        bufs[b], sems[b]).start()
            return carry
        lax.fori_loop(0, N_CHUNKS // NBUF, outer, 0)

        lane16 = lane * L

        @plsc.parallel_loop(0, nbins, step=L, unroll=2)
        def _(g):
            basei = g * L
            acc = zeros16
            for l in range(L):
                acc = acc + plsc.load_gather(histv, [basei + lane16 + l])
            histr[pl.ds(g, L)] = acc

        pltpu.sync_copy(histr, out_hbm.at[wid])

    return functools.partial(
        pl.kernel, body,
        out_type=jax.ShapeDtypeStruct((NW, nbins), jnp.int32),
        mesh=mesh, scratch_types=scratch,
        compiler_params=pltpu.CompilerParams(
            needs_layout_passes=False, use_tc_tiling_on_sc=False),
        name=f"kwta_hist{stage}")()


def _tri(n, lower):
    i = lax.broadcasted_iota(jnp.int32, (n, n), 0)
    j = lax.broadcasted_iota(jnp.int32, (n, n), 1)
    return ((i < j) if lower else (i > j)).astype(jnp.float32)


def _matmul(a, b):
    return lax.dot_general(a, b, (((1,), (0,)), ((), ())),
                           precision=lax.Precision.HIGHEST,
                           preferred_element_type=jnp.float32)


def _select_body(stage, *refs):
    nbins = 4096 if stage < 3 else 256
    nch = nbins // 128
    if stage == 1:
        h_ref, pref_out, carry_out = refs
        carry_ref = None
    elif stage == 2:
        h_ref, carry_ref, pref_out, carry_out = refs
    else:
        h_ref, carry_ref, t_out = refs

    h = h_ref[...].astype(jnp.float32)
    hr = jnp.sum(h.reshape(B, NW // B, nbins), axis=1)
    h3 = hr.reshape(B, nch, 128)
    cs = jnp.sum(h3, axis=2)

    wl_up = _matmul(h3.reshape(B * nch, 128), _tri(128, False)
                    ).reshape(B, nch, 128)
    wl_dn = _matmul(h3.reshape(B * nch, 128), _tri(128, True)
                    ).reshape(B, nch, 128)

    if stage == 1:
        ic = lax.broadcasted_iota(jnp.int32, (nch, nch), 0)
        jc = lax.broadcasted_iota(jnp.int32, (nch, nch), 1)
        posrow, poscol = ic < nch // 2, jc < nch // 2
        mc = ((poscol & posrow & (ic > jc))
              | ((~poscol) & (posrow | ((~posrow) & (ic < jc))))
              ).astype(jnp.float32)
        gc = _matmul(cs, mc)
        chunkpos = (lax.broadcasted_iota(jnp.int32, (1, nch, 1), 1)
                    < nch // 2)
        g = gc[:, :, None] + jnp.where(chunkpos, wl_up, wl_dn)
        kv = jnp.full((B, 1, 1), float(K), jnp.float32)
    else:
        gc_up = _matmul(cs, _tri(nch, False))
        gc_dn = _matmul(cs, _tri(nch, True))
        prev = carry_ref[...][:, 0:1]
        negrow = (prev >= (2048 if stage == 2 else 0x800000))[:, :, None]
        g = jnp.where(negrow, gc_dn[:, :, None] + wl_dn,
                      gc_up[:, :, None] + wl_up)
        kv = carry_ref[...][:, 1:2].astype(jnp.float32)[:, :, None]

    cond = (g < kv) & (g + h3 >= kv)
    bidx = (lax.broadcasted_iota(jnp.int32, (B, nch, 128), 1) * 128
            + lax.broadcasted_iota(jnp.int32, (B, nch, 128), 2)
            ).astype(jnp.float32)
    bsel = jnp.sum(jnp.sum(jnp.where(cond, bidx, 0.0), axis=2),
                   axis=1, keepdims=True)
    kres = jnp.sum(jnp.sum(jnp.where(cond, kv - g, 0.0), axis=2),
                   axis=1, keepdims=True)
    bi = bsel.astype(jnp.int32)
    ki = kres.astype(jnp.int32)

    if stage == 3:
        ut = carry_ref[...][:, 0:1] * 256 + bi
        t_out[...] = jnp.broadcast_to(
            lax.bitcast_convert_type(ut, jnp.float32), (B, 128))
        return
    if stage == 1:
        newraw = bi
        sext = jnp.where(bi >= 2048, bi - 4096, bi)
    else:
        newraw = carry_ref[...][:, 0:1] * 4096 + bi
        sext = jnp.where(newraw >= 0x800000, newraw - 0x1000000, newraw)

    pref_out[...] = jnp.broadcast_to(
        sext[:, None, :], (B, NW // B, 128)).reshape(NW, 128)
    ci = lax.broadcasted_iota(jnp.int32, (B, 128), 1)
    carry_out[...] = jnp.where(
        ci == 0, jnp.broadcast_to(newraw, (B, 128)),
        jnp.where(ci == 1, jnp.broadcast_to(ki, (B, 128)), 0))


def _select(stage, *args):
    if stage < 3:
        out_shape = (jax.ShapeDtypeStruct((NW, 128), jnp.int32),
                     jax.ShapeDtypeStruct((B, 128), jnp.int32))
    else:
        out_shape = jax.ShapeDtypeStruct((B, 128), jnp.float32)
    return pl.pallas_call(
        functools.partial(_select_body, stage),
        out_shape=out_shape,
        name=f"kwta_select{stage}")(*args)


_MC = 24
assert C_ % _MC == 0


def _mask_body(t_ref, x_ref, o_ref):
    tv = t_ref[pl.program_id(0), 0]
    xb = x_ref[...]
    o_ref[...] = jnp.where(xb >= tv, xb, 0.0)


def _mask(x, t):
    return pl.pallas_call(
        _mask_body,
        grid=(B, C_ // _MC),
        in_specs=[
            pl.BlockSpec((B, 128), lambda r, c: (0, 0)),
            pl.BlockSpec((1, _MC, H_, W_), lambda r, c: (r, c, 0, 0)),
        ],
        out_specs=pl.BlockSpec((1, _MC, H_, W_), lambda r, c: (r, c, 0, 0)),
        out_shape=jax.ShapeDtypeStruct((B, C_, H_, W_), jnp.float32),
        name="kwta_mask")(t, x)


def kernel(x):
    xf = x.reshape(-1)
    h1 = _make_hist_pass(1)(xf)
    p1, c1 = _select(1, h1)
    h2 = _make_hist_pass(2)(xf, p1)
    p2, c2 = _select(2, h2, c1)
    h3 = _make_hist_pass(3)(xf, p2)
    t = _select(3, h3, c2)
    return _mask(x, t)
